# baseline (speedup 1.0000x reference)
import functools

import jax
import jax.numpy as jnp
from jax import lax
from jax.experimental import pallas as pl
from jax.experimental.pallas import tpu as pltpu
from jax.experimental.pallas import tpu_sc as plsc

DEPTH = 5
NC, NS = 2, 16
NW = NC * NS
MAX_IN = 4


def _as_bf16(x_i32):
    m, w = x_i32.shape
    return lax.bitcast_convert_type(x_i32, jnp.bfloat16).reshape(m, 2 * w)


def _as_i32(x_bf16):
    m, h = x_bf16.shape
    return lax.bitcast_convert_type(x_bf16.reshape(m, h // 2, 2), jnp.int32)


def _gather_sc(table, idx_flat, m_rows, chunk):
    n_rows, width = table.shape
    total_chunks = m_rows // chunk
    rows_pc = MAX_IN * chunk
    assert m_rows % chunk == 0 and chunk % 8 == 0
    assert rows_pc <= 128
    mesh = plsc.VectorSubcoreMesh(core_axis_name="c", subcore_axis_name="s",
                                  num_cores=NC, num_subcores=NS)

    @functools.partial(
        pl.kernel,
        out_type=jax.ShapeDtypeStruct((MAX_IN * m_rows, width), jnp.int32),
        mesh=mesh,
        scratch_types=[
            pltpu.VMEM((rows_pc,), jnp.int32),
            pltpu.VMEM((rows_pc,), jnp.int32),
            pltpu.VMEM((rows_pc, width), jnp.int32),
            pltpu.VMEM((rows_pc, width), jnp.int32),
            pltpu.SemaphoreType.DMA,
            pltpu.SemaphoreType.DMA,
        ],
    )
    def gather_kernel(table_hbm, idx_hbm, out_hbm, idx0, idx1, rows0, rows1,
                      sem0, sem1):
        wid = lax.axis_index("s") * NC + lax.axis_index("c")
        n_mine = (total_chunks - wid + NW - 1) // NW
        idx_b, rows_b, sem_b = (idx0, idx1), (rows0, rows1), (sem0, sem1)

        def start_gather(i, b):
            ci = wid + i * NW
            pltpu.sync_copy(
                idx_hbm.at[pl.ds(rows_pc * ci, rows_pc)], idx_b[b])
            pltpu.async_copy(table_hbm.at[idx_b[b]], rows_b[b], sem_b[b])

        def finish(i, b):
            ci = wid + i * NW
            pltpu.make_async_copy(table_hbm.at[idx_b[b]], rows_b[b],
                                  sem_b[b]).wait()
            pltpu.sync_copy(rows_b[b],
                            out_hbm.at[pl.ds(ci * rows_pc, rows_pc)])

        @pl.when(n_mine > 0)
        def _prime():
            start_gather(0, 0)

        def pair(p, carry):
            i0, i1 = 2 * p, 2 * p + 1

            @pl.when(i1 < n_mine)
            def _s1():
                start_gather(i1, 1)

            finish(i0, 0)

            @pl.when(i1 + 1 < n_mine)
            def _s0():
                start_gather(i1 + 1, 0)

            @pl.when(i1 < n_mine)
            def _f1():
                finish(i1, 1)

            return carry

        lax.fori_loop(0, (n_mine + 1) // 2, pair, 0)

    return gather_kernel(table, idx_flat)


def _sum4(g):
    m4, h = g.shape
    gs = g.reshape(m4 // MAX_IN, MAX_IN, h)
    return (gs[:, 0] + gs[:, 1]) + (gs[:, 2] + gs[:, 3])


def _proj_body(x_ref, w_ref, inp_ref, msg_ref):
    acc = jnp.dot(x_ref[...], w_ref[...], preferred_element_type=jnp.float32)
    inp_ref[...] = acc
    msg_ref[...] = jnp.maximum(acc, 0.0).astype(jnp.bfloat16)


def _step_body(g_ref, inp_ref, w_ref, msg_ref):
    h = inp_ref[...] + jnp.dot(_sum4(g_ref[...]), w_ref[...],
                               preferred_element_type=jnp.float32)
    msg_ref[...] = jnp.maximum(h, 0.0).astype(jnp.bfloat16)


def _step_last_body(g_ref, inp_ref, w_ref, h_ref):
    h = inp_ref[...] + jnp.dot(_sum4(g_ref[...]), w_ref[...],
                               preferred_element_type=jnp.float32)
    h_ref[...] = h.astype(jnp.bfloat16)


def _tail_body(apm, af_ref, msgs_ref, gf_ref, woa_ref, wom_ref, bo_ref,
               w1g_ref, w1m_ref, b1_ref, w2_ref, b2_ref, w3t_ref, out_ref):
    msgs = _sum4(msgs_ref[...])
    hidden = jnp.maximum(
        jnp.dot(af_ref[...], woa_ref[...], preferred_element_type=jnp.float32)
        + jnp.dot(msgs, wom_ref[...], preferred_element_type=jnp.float32)
        + bo_ref[...], 0.0)
    n_atoms_blk, hid = hidden.shape
    mols = n_atoms_blk // apm
    mol = jnp.mean(hidden.reshape(mols, apm, hid), axis=1)
    h1 = jnp.maximum(
        jnp.dot(mol, w1m_ref[...], preferred_element_type=jnp.float32)
        + jnp.dot(gf_ref[...], w1g_ref[...],
                  preferred_element_type=jnp.float32)
        + b1_ref[...], 0.0)
    h2 = jnp.maximum(
        jnp.dot(h1, w2_ref[...], preferred_element_type=jnp.float32)
        + b2_ref[...], 0.0)
    out_ref[...] = jnp.sum(h2 * w3t_ref[...], axis=1, keepdims=True)


def kernel(atom_features, f_ini_atoms_bonds, atom_to_incoming_bonds, mapping,
           global_features, W_i, W_h, W_o, b_o, W_ffn1, b_ffn1, W_ffn2,
           b_ffn2, W_ffn3, b_ffn3):
    n_atoms, atom_f = atom_features.shape
    n_bonds, concat_f = f_ini_atoms_bonds.shape
    n_mols, gf_dim = global_features.shape
    hid = W_h.shape[0]
    apm = n_atoms // n_mols

    mb = 800
    inp, message = pl.pallas_call(
        _proj_body,
        grid=(n_bonds // mb,),
        in_specs=[
            pl.BlockSpec((mb, concat_f), lambda i: (i, 0)),
            pl.BlockSpec((concat_f, hid), lambda i: (0, 0)),
        ],
        out_specs=[
            pl.BlockSpec((mb, hid), lambda i: (i, 0)),
            pl.BlockSpec((mb, hid), lambda i: (i, 0)),
        ],
        out_shape=[
            jax.ShapeDtypeStruct((n_bonds, hid), jnp.float32),
            jax.ShapeDtypeStruct((n_bonds, hid), jnp.bfloat16),
        ],
    )(f_ini_atoms_bonds, W_i)

    map_flat = mapping.reshape(-1).astype(jnp.int32)
    W_h_bf = W_h.astype(jnp.bfloat16)

    def make_step(body, out_dtype):
        return pl.pallas_call(
            body,
            grid=(n_bonds // mb,),
            in_specs=[
                pl.BlockSpec((MAX_IN * mb, hid), lambda i: (i, 0)),
                pl.BlockSpec((mb, hid), lambda i: (i, 0)),
                pl.BlockSpec((hid, hid), lambda i: (0, 0)),
            ],
            out_specs=pl.BlockSpec((mb, hid), lambda i: (i, 0)),
            out_shape=jax.ShapeDtypeStruct((n_bonds, hid), out_dtype),
        )

    step_call = make_step(_step_body, jnp.bfloat16)
    step_last = make_step(_step_last_body, jnp.bfloat16)
    for _ in range(1, DEPTH - 1):
        gathered = _gather_sc(_as_i32(message), map_flat, n_bonds, 32)
        message = step_call(_as_bf16(gathered), inp, W_h_bf)
    gathered = _gather_sc(_as_i32(message), map_flat, n_bonds, 32)
    h_message = step_last(_as_bf16(gathered), inp, W_h_bf)

    atib_flat = atom_to_incoming_bonds.reshape(-1).astype(jnp.int32)
    msgs_to_atoms = _gather_sc(_as_i32(h_message), atib_flat, n_atoms, 16)

    mol_blk = 40
    atoms_blk = mol_blk * apm
    out = pl.pallas_call(
        functools.partial(_tail_body, apm),
        grid=(n_mols // mol_blk,),
        in_specs=[
            pl.BlockSpec((atoms_blk, atom_f), lambda i: (i, 0)),
            pl.BlockSpec((MAX_IN * atoms_blk, hid), lambda i: (i, 0)),
            pl.BlockSpec((mol_blk, gf_dim), lambda i: (i, 0)),
            pl.BlockSpec((atom_f, hid), lambda i: (0, 0)),
            pl.BlockSpec((hid, hid), lambda i: (0, 0)),
            pl.BlockSpec((1, hid), lambda i: (0, 0)),
            pl.BlockSpec((gf_dim, hid), lambda i: (0, 0)),
            pl.BlockSpec((hid, hid), lambda i: (0, 0)),
            pl.BlockSpec((1, hid), lambda i: (0, 0)),
            pl.BlockSpec((hid, hid), lambda i: (0, 0)),
            pl.BlockSpec((1, hid), lambda i: (0, 0)),
            pl.BlockSpec((1, hid), lambda i: (0, 0)),
        ],
        out_specs=pl.BlockSpec((mol_blk, 1), lambda i: (i, 0)),
        out_shape=jax.ShapeDtypeStruct((n_mols, 1), jnp.float32),
    )(atom_features, _as_bf16(msgs_to_atoms), global_features,
      W_o[:atom_f], W_o[atom_f:], b_o.reshape(1, hid),
      W_ffn1[hid:], W_ffn1[:hid], b_ffn1.reshape(1, hid),
      W_ffn2, b_ffn2.reshape(1, hid), W_ffn3.reshape(1, hid))
    return out + b_ffn3


# speedup vs baseline: 4.1051x; 4.1051x over previous
import functools

import jax
import jax.numpy as jnp
from jax import lax
from jax.experimental import pallas as pl
from jax.experimental.pallas import tpu as pltpu
from jax.experimental.pallas import tpu_sc as plsc

DEPTH = 5
NC, NS = 2, 16
NW = NC * NS
MAX_IN = 4


def _gather_sum_sc(table, idx_flat, m_rows, chunk):
    n_rows, width = table.shape
    total_chunks = m_rows // chunk
    rows_pc = MAX_IN * chunk
    assert m_rows % chunk == 0 and chunk % 8 == 0
    assert rows_pc <= 128
    mesh = plsc.VectorSubcoreMesh(core_axis_name="c", subcore_axis_name="s",
                                  num_cores=NC, num_subcores=NS)

    @functools.partial(
        pl.kernel,
        out_type=jax.ShapeDtypeStruct((m_rows, width), jnp.float32),
        mesh=mesh,
        scratch_types=[
            pltpu.VMEM((rows_pc,), jnp.int32),
            pltpu.VMEM((rows_pc,), jnp.int32),
            pltpu.VMEM((rows_pc, width), jnp.float32),
            pltpu.VMEM((rows_pc, width), jnp.float32),
            pltpu.VMEM((chunk, width), jnp.float32),
            pltpu.SemaphoreType.DMA,
            pltpu.SemaphoreType.DMA,
        ],
    )
    def gather_kernel(table_hbm, idx_hbm, out_hbm, idx0, idx1, rows0, rows1,
                      acc, sem0, sem1):
        wid = lax.axis_index("s") * NC + lax.axis_index("c")
        n_mine = (total_chunks - wid + NW - 1) // NW
        idx_b, rows_b, sem_b = (idx0, idx1), (rows0, rows1), (sem0, sem1)

        def start_gather(i, b):
            ci = wid + i * NW
            pltpu.sync_copy(
                idx_hbm.at[pl.ds(rows_pc * ci, rows_pc)], idx_b[b])
            pltpu.async_copy(table_hbm.at[idx_b[b]], rows_b[b], sem_b[b])

        def finish(i, b):
            ci = wid + i * NW
            pltpu.make_async_copy(table_hbm.at[idx_b[b]], rows_b[b],
                                  sem_b[b]).wait()
            rows = rows_b[b]

            def row_body(r, c2):
                for s in range(width // 16):
                    sl = pl.ds(s * 16, 16)
                    vals = [rows[MAX_IN * r + j, sl] for j in range(MAX_IN)]
                    acc[r, sl] = (vals[0] + vals[1]) + (vals[2] + vals[3])
                return c2

            lax.fori_loop(0, chunk, row_body, 0)
            pltpu.sync_copy(acc, out_hbm.at[pl.ds(ci * chunk, chunk)])

        @pl.when(n_mine > 0)
        def _prime():
            start_gather(0, 0)

        def pair(p, carry):
            i0, i1 = 2 * p, 2 * p + 1

            @pl.when(i1 < n_mine)
            def _s1():
                start_gather(i1, 1)

            finish(i0, 0)

            @pl.when(i1 + 1 < n_mine)
            def _s0():
                start_gather(i1 + 1, 0)

            @pl.when(i1 < n_mine)
            def _f1():
                finish(i1, 1)

            return carry

        lax.fori_loop(0, (n_mine + 1) // 2, pair, 0)

    return gather_kernel(table, idx_flat)


def _proj_body(x_ref, w_ref, inp_ref, msg_ref):
    acc = jnp.dot(x_ref[...], w_ref[...], preferred_element_type=jnp.float32)
    inp_ref[...] = acc
    msg_ref[...] = jnp.maximum(acc, 0.0)


def _step_body(g_ref, inp_ref, w_ref, msg_ref):
    h = inp_ref[...] + jnp.dot(g_ref[...], w_ref[...],
                               preferred_element_type=jnp.float32)
    msg_ref[...] = jnp.maximum(h, 0.0)


def _step_last_body(g_ref, inp_ref, w_ref, h_ref):
    h_ref[...] = inp_ref[...] + jnp.dot(g_ref[...], w_ref[...],
                                        preferred_element_type=jnp.float32)


def _tail_body(apm, af_ref, msgs_ref, gf_ref, woa_ref, wom_ref, bo_ref,
               w1g_ref, w1m_ref, b1_ref, w2_ref, b2_ref, w3t_ref, out_ref):
    hidden = jnp.maximum(
        jnp.dot(af_ref[...], woa_ref[...], preferred_element_type=jnp.float32)
        + jnp.dot(msgs_ref[...], wom_ref[...],
                  preferred_element_type=jnp.float32)
        + bo_ref[...], 0.0)
    n_atoms_blk, hid = hidden.shape
    mols = n_atoms_blk // apm
    mol = jnp.mean(hidden.reshape(mols, apm, hid), axis=1)
    h1 = jnp.maximum(
        jnp.dot(mol, w1m_ref[...], preferred_element_type=jnp.float32)
        + jnp.dot(gf_ref[...], w1g_ref[...],
                  preferred_element_type=jnp.float32)
        + b1_ref[...], 0.0)
    h2 = jnp.maximum(
        jnp.dot(h1, w2_ref[...], preferred_element_type=jnp.float32)
        + b2_ref[...], 0.0)
    out_ref[...] = jnp.sum(h2 * w3t_ref[...], axis=1, keepdims=True)


def kernel(atom_features, f_ini_atoms_bonds, atom_to_incoming_bonds, mapping,
           global_features, W_i, W_h, W_o, b_o, W_ffn1, b_ffn1, W_ffn2,
           b_ffn2, W_ffn3, b_ffn3):
    n_atoms, atom_f = atom_features.shape
    n_bonds, concat_f = f_ini_atoms_bonds.shape
    n_mols, gf_dim = global_features.shape
    hid = W_h.shape[0]
    apm = n_atoms // n_mols

    mb = 800
    inp, message = pl.pallas_call(
        _proj_body,
        grid=(n_bonds // mb,),
        in_specs=[
            pl.BlockSpec((mb, concat_f), lambda i: (i, 0)),
            pl.BlockSpec((concat_f, hid), lambda i: (0, 0)),
        ],
        out_specs=[
            pl.BlockSpec((mb, hid), lambda i: (i, 0)),
            pl.BlockSpec((mb, hid), lambda i: (i, 0)),
        ],
        out_shape=[
            jax.ShapeDtypeStruct((n_bonds, hid), jnp.float32),
            jax.ShapeDtypeStruct((n_bonds, hid), jnp.float32),
        ],
    )(f_ini_atoms_bonds, W_i)

    map_flat = mapping.reshape(-1).astype(jnp.int32)

    def make_step(body):
        return pl.pallas_call(
            body,
            grid=(n_bonds // mb,),
            in_specs=[
                pl.BlockSpec((mb, hid), lambda i: (i, 0)),
                pl.BlockSpec((mb, hid), lambda i: (i, 0)),
                pl.BlockSpec((hid, hid), lambda i: (0, 0)),
            ],
            out_specs=pl.BlockSpec((mb, hid), lambda i: (i, 0)),
            out_shape=jax.ShapeDtypeStruct((n_bonds, hid), jnp.float32),
        )

    step_call = make_step(_step_body)
    step_last = make_step(_step_last_body)
    for _ in range(1, DEPTH - 1):
        gathered = _gather_sum_sc(message, map_flat, n_bonds, 16)
        message = step_call(gathered, inp, W_h)
    gathered = _gather_sum_sc(message, map_flat, n_bonds, 16)
    h_message = step_last(gathered, inp, W_h)

    atib_flat = atom_to_incoming_bonds.reshape(-1).astype(jnp.int32)
    msgs_to_atoms = _gather_sum_sc(h_message, atib_flat, n_atoms, 16)

    mol_blk = 40
    atoms_blk = mol_blk * apm
    out = pl.pallas_call(
        functools.partial(_tail_body, apm),
        grid=(n_mols // mol_blk,),
        in_specs=[
            pl.BlockSpec((atoms_blk, atom_f), lambda i: (i, 0)),
            pl.BlockSpec((atoms_blk, hid), lambda i: (i, 0)),
            pl.BlockSpec((mol_blk, gf_dim), lambda i: (i, 0)),
            pl.BlockSpec((atom_f, hid), lambda i: (0, 0)),
            pl.BlockSpec((hid, hid), lambda i: (0, 0)),
            pl.BlockSpec((1, hid), lambda i: (0, 0)),
            pl.BlockSpec((gf_dim, hid), lambda i: (0, 0)),
            pl.BlockSpec((hid, hid), lambda i: (0, 0)),
            pl.BlockSpec((1, hid), lambda i: (0, 0)),
            pl.BlockSpec((hid, hid), lambda i: (0, 0)),
            pl.BlockSpec((1, hid), lambda i: (0, 0)),
            pl.BlockSpec((1, hid), lambda i: (0, 0)),
        ],
        out_specs=pl.BlockSpec((mol_blk, 1), lambda i: (i, 0)),
        out_shape=jax.ShapeDtypeStruct((n_mols, 1), jnp.float32),
    )(atom_features, msgs_to_atoms, global_features,
      W_o[:atom_f], W_o[atom_f:], b_o.reshape(1, hid),
      W_ffn1[hid:], W_ffn1[:hid], b_ffn1.reshape(1, hid),
      W_ffn2, b_ffn2.reshape(1, hid), W_ffn3.reshape(1, hid))
    return out + b_ffn3


# speedup vs baseline: 8.4815x; 2.0661x over previous
import functools

import jax
import jax.numpy as jnp
from jax import lax
from jax.experimental import pallas as pl
from jax.experimental.pallas import tpu as pltpu
from jax.experimental.pallas import tpu_sc as plsc

DEPTH = 5
NC, NS = 2, 16
NW = NC * NS
MAX_IN = 4
_MASK_HI = -65536


def _pack_bf16(x):
    half = x.shape[1] // 2
    xb = x.astype(jnp.bfloat16)
    lo = lax.convert_element_type(
        lax.bitcast_convert_type(xb[:, :half], jnp.uint16), jnp.uint32)
    hi = lax.convert_element_type(
        lax.bitcast_convert_type(xb[:, half:], jnp.uint16), jnp.uint32)
    return lax.bitcast_convert_type(lo | (hi << 16), jnp.int32)


def _unpack_f32(w):
    lo = lax.bitcast_convert_type(w << 16, jnp.float32)
    hi = lax.bitcast_convert_type(w & _MASK_HI, jnp.float32)
    return lo, hi


def _sum_slots(g, m):
    return ((g[0 * m:1 * m] + g[1 * m:2 * m])
            + (g[2 * m:3 * m] + g[3 * m:4 * m]))


def _slot_major_idx(idx2d, blk):
    m = idx2d.shape[0]
    return (idx2d.reshape(m // blk, blk, MAX_IN)
            .transpose(0, 2, 1).reshape(-1).astype(jnp.int32))


def _gather_sc(table, idx_flat, rows_pc=128):
    n_rows, width = table.shape
    total_rows = idx_flat.shape[0]
    total_chunks = total_rows // rows_pc
    assert total_rows % rows_pc == 0 and rows_pc <= 128
    mesh = plsc.VectorSubcoreMesh(core_axis_name="c", subcore_axis_name="s",
                                  num_cores=NC, num_subcores=NS)

    @functools.partial(
        pl.kernel,
        out_type=jax.ShapeDtypeStruct((total_rows, width), jnp.int32),
        mesh=mesh,
        scratch_types=[
            pltpu.VMEM((rows_pc,), jnp.int32),
            pltpu.VMEM((rows_pc,), jnp.int32),
            pltpu.VMEM((rows_pc, width), jnp.int32),
            pltpu.VMEM((rows_pc, width), jnp.int32),
            pltpu.SemaphoreType.DMA,
            pltpu.SemaphoreType.DMA,
        ],
    )
    def gather_kernel(table_hbm, idx_hbm, out_hbm, idx0, idx1, rows0, rows1,
                      sem0, sem1):
        wid = lax.axis_index("s") * NC + lax.axis_index("c")
        n_mine = (total_chunks - wid + NW - 1) // NW
        idx_b, rows_b, sem_b = (idx0, idx1), (rows0, rows1), (sem0, sem1)

        def start_gather(i, b):
            ci = wid + i * NW
            pltpu.sync_copy(
                idx_hbm.at[pl.ds(rows_pc * ci, rows_pc)], idx_b[b])
            pltpu.async_copy(table_hbm.at[idx_b[b]], rows_b[b], sem_b[b])

        def finish(i, b):
            ci = wid + i * NW
            pltpu.make_async_copy(table_hbm.at[idx_b[b]], rows_b[b],
                                  sem_b[b]).wait()
            pltpu.sync_copy(rows_b[b],
                            out_hbm.at[pl.ds(ci * rows_pc, rows_pc)])

        @pl.when(n_mine > 0)
        def _prime():
            start_gather(0, 0)

        def pair(p, carry):
            i0, i1 = 2 * p, 2 * p + 1

            @pl.when(i1 < n_mine)
            def _s1():
                start_gather(i1, 1)

            finish(i0, 0)

            @pl.when(i1 + 1 < n_mine)
            def _s0():
                start_gather(i1 + 1, 0)

            @pl.when(i1 < n_mine)
            def _f1():
                finish(i1, 1)

            return carry

        lax.fori_loop(0, (n_mine + 1) // 2, pair, 0)

    return gather_kernel(table, idx_flat)


def _proj_body(x_ref, w_ref, inp_ref, msg_ref):
    acc = jnp.dot(x_ref[...], w_ref[...], preferred_element_type=jnp.float32)
    inp_ref[...] = acc
    msg_ref[...] = _pack_bf16(jnp.maximum(acc, 0.0))


def _step_core(g_ref, inp_ref, wt_ref, wb_ref):
    m = inp_ref.shape[0]
    lo, hi = _unpack_f32(g_ref[...])
    lo_s = _sum_slots(lo, m)
    hi_s = _sum_slots(hi, m)
    return (inp_ref[...]
            + jnp.dot(lo_s, wt_ref[...], preferred_element_type=jnp.float32)
            + jnp.dot(hi_s, wb_ref[...], preferred_element_type=jnp.float32))


def _step_body(g_ref, inp_ref, wt_ref, wb_ref, msg_ref):
    h = _step_core(g_ref, inp_ref, wt_ref, wb_ref)
    msg_ref[...] = _pack_bf16(jnp.maximum(h, 0.0))


def _step_last_body(g_ref, inp_ref, wt_ref, wb_ref, h_ref):
    h_ref[...] = _pack_bf16(_step_core(g_ref, inp_ref, wt_ref, wb_ref))


def _tail_body(apm, af_ref, msgs_ref, gf_ref, woa_ref, womt_ref, womb_ref,
               bo_ref, w1g_ref, w1m_ref, b1_ref, w2_ref, b2_ref, w3t_ref,
               out_ref):
    n_atoms_blk = af_ref.shape[0]
    lo, hi = _unpack_f32(msgs_ref[...])
    lo_s = _sum_slots(lo, n_atoms_blk)
    hi_s = _sum_slots(hi, n_atoms_blk)
    hidden = jnp.maximum(
        jnp.dot(af_ref[...], woa_ref[...], preferred_element_type=jnp.float32)
        + jnp.dot(lo_s, womt_ref[...], preferred_element_type=jnp.float32)
        + jnp.dot(hi_s, womb_ref[...], preferred_element_type=jnp.float32)
        + bo_ref[...], 0.0)
    hid = hidden.shape[1]
    mols = n_atoms_blk // apm
    mol = jnp.mean(hidden.reshape(mols, apm, hid), axis=1)
    h1 = jnp.maximum(
        jnp.dot(mol, w1m_ref[...], preferred_element_type=jnp.float32)
        + jnp.dot(gf_ref[...], w1g_ref[...],
                  preferred_element_type=jnp.float32)
        + b1_ref[...], 0.0)
    h2 = jnp.maximum(
        jnp.dot(h1, w2_ref[...], preferred_element_type=jnp.float32)
        + b2_ref[...], 0.0)
    out_ref[...] = jnp.sum(h2 * w3t_ref[...], axis=1, keepdims=True)


def kernel(atom_features, f_ini_atoms_bonds, atom_to_incoming_bonds, mapping,
           global_features, W_i, W_h, W_o, b_o, W_ffn1, b_ffn1, W_ffn2,
           b_ffn2, W_ffn3, b_ffn3):
    n_atoms, atom_f = atom_features.shape
    n_bonds, concat_f = f_ini_atoms_bonds.shape
    n_mols, gf_dim = global_features.shape
    hid = W_h.shape[0]
    half = hid // 2
    apm = n_atoms // n_mols

    mb = 800
    inp, message = pl.pallas_call(
        _proj_body,
        grid=(n_bonds // mb,),
        in_specs=[
            pl.BlockSpec((mb, concat_f), lambda i: (i, 0)),
            pl.BlockSpec((concat_f, hid), lambda i: (0, 0)),
        ],
        out_specs=[
            pl.BlockSpec((mb, hid), lambda i: (i, 0)),
            pl.BlockSpec((mb, half), lambda i: (i, 0)),
        ],
        out_shape=[
            jax.ShapeDtypeStruct((n_bonds, hid), jnp.float32),
            jax.ShapeDtypeStruct((n_bonds, half), jnp.int32),
        ],
    )(f_ini_atoms_bonds, W_i)

    map_flat = _slot_major_idx(mapping, mb)
    wh_top, wh_bot = W_h[:half], W_h[half:]

    def make_step(body):
        return pl.pallas_call(
            body,
            grid=(n_bonds // mb,),
            in_specs=[
                pl.BlockSpec((MAX_IN * mb, half), lambda i: (i, 0)),
                pl.BlockSpec((mb, hid), lambda i: (i, 0)),
                pl.BlockSpec((half, hid), lambda i: (0, 0)),
                pl.BlockSpec((half, hid), lambda i: (0, 0)),
            ],
            out_specs=pl.BlockSpec((mb, half), lambda i: (i, 0)),
            out_shape=jax.ShapeDtypeStruct((n_bonds, half), jnp.int32),
        )

    step_call = make_step(_step_body)
    step_last = make_step(_step_last_body)
    for _ in range(1, DEPTH - 1):
        gathered = _gather_sc(message, map_flat)
        message = step_call(gathered, inp, wh_top, wh_bot)
    gathered = _gather_sc(message, map_flat)
    h_message = step_last(gathered, inp, wh_top, wh_bot)

    mol_blk = 40
    atoms_blk = mol_blk * apm
    atib_flat = _slot_major_idx(atom_to_incoming_bonds, atoms_blk)
    msgs_to_atoms = _gather_sc(h_message, atib_flat, rows_pc=80)

    w_om = W_o[atom_f:]
    out = pl.pallas_call(
        functools.partial(_tail_body, apm),
        grid=(n_mols // mol_blk,),
        in_specs=[
            pl.BlockSpec((atoms_blk, atom_f), lambda i: (i, 0)),
            pl.BlockSpec((MAX_IN * atoms_blk, half), lambda i: (i, 0)),
            pl.BlockSpec((mol_blk, gf_dim), lambda i: (i, 0)),
            pl.BlockSpec((atom_f, hid), lambda i: (0, 0)),
            pl.BlockSpec((half, hid), lambda i: (0, 0)),
            pl.BlockSpec((half, hid), lambda i: (0, 0)),
            pl.BlockSpec((1, hid), lambda i: (0, 0)),
            pl.BlockSpec((gf_dim, hid), lambda i: (0, 0)),
            pl.BlockSpec((hid, hid), lambda i: (0, 0)),
            pl.BlockSpec((1, hid), lambda i: (0, 0)),
            pl.BlockSpec((hid, hid), lambda i: (0, 0)),
            pl.BlockSpec((1, hid), lambda i: (0, 0)),
            pl.BlockSpec((1, hid), lambda i: (0, 0)),
        ],
        out_specs=pl.BlockSpec((mol_blk, 1), lambda i: (i, 0)),
        out_shape=jax.ShapeDtypeStruct((n_mols, 1), jnp.float32),
    )(atom_features, msgs_to_atoms, global_features,
      W_o[:atom_f], w_om[:half], w_om[half:], b_o.reshape(1, hid),
      W_ffn1[hid:], W_ffn1[:hid], b_ffn1.reshape(1, hid),
      W_ffn2, b_ffn2.reshape(1, hid), W_ffn3.reshape(1, hid))
    return out + b_ffn3
